# Initial kernel scaffold; baseline (speedup 1.0000x reference)
import jax
import jax.numpy as jnp
from jax.experimental import pallas as pl


def kernel(t, x, d_base, emb_W, emb_b, edge_W1, edge_b1, edge_W2, edge_b2, node_W1, node_b1, node_W2, node_b2, coord_W1, coord_b1, coord_W2, att_W, att_b):
    raise NotImplementedError("write your pallas kernel here")


# speedup vs baseline: 9.5997x; 9.5997x over previous
import jax
import jax.numpy as jnp
from jax import lax
from jax.experimental import pallas as pl
from jax.experimental.pallas import tpu as pltpu

_N = 55
_P = 56
_H = 64
_L = 5
_PP = _P * _P
_CR = 3.0


def _pair_diffs(coord_c, coord_r):
    out = []
    for k in range(3):
        col = coord_c[:, k:k + 1]
        row = coord_r[k:k + 1, :]
        out.append(col[:, None, :] - row[:, :, None])
    return out


def _fwd_kernel(t_ref, d_ref, x_ref, xt_ref, embW_ref, embb_ref,
                eW1a_ref, eW1b_ref, ewr_ref, ewe_ref, eb1_ref,
                eW2_ref, eb2_ref,
                nW1a_ref, nW1b_ref, nb1_ref, nW2_ref, nb2_ref,
                cW1_ref, cb1_ref, cW2r_ref,
                aWr_ref, ab_ref,
                out_ref):
    f32 = jnp.float32
    silu = jax.nn.silu

    bi = lax.broadcasted_iota(jnp.int32, (_P, _P, 1), 0)
    bj = lax.broadcasted_iota(jnp.int32, (_P, _P, 1), 1)
    mask_agg = ((bi != bj) & (bj < _N)).astype(f32).reshape(_PP, 1)
    mask_j3 = (bj < _N).astype(f32)

    x0c = jnp.concatenate(
        [x_ref[0], jnp.zeros((_P - _N, 3), f32)], axis=0)
    x0r = jnp.concatenate(
        [xt_ref[0], jnp.zeros((3, _P - _N), f32)], axis=1)

    h0 = (t_ref[0] * embW_ref[0:1, :] + d_ref[0] * embW_ref[1:2, :]
          + embb_ref[...])
    h = jnp.broadcast_to(h0, (_P, _H))

    d3 = _pair_diffs(x0c, x0r)
    rad3 = d3[0] * d3[0] + d3[1] * d3[1] + d3[2] * d3[2]
    ea3 = rad3

    coord_c, coord_r = x0c, x0r
    for l in range(_L):
        if l:
            d3 = _pair_diffs(coord_c, coord_r)
            rad3 = d3[0] * d3[0] + d3[1] * d3[1] + d3[2] * d3[2]
        inv3 = 1.0 / (jnp.sqrt(rad3) + 1.0)

        preI = jnp.dot(h, eW1a_ref[l], preferred_element_type=f32) \
            + eb1_ref[l]
        preJ = jnp.dot(h, eW1b_ref[l], preferred_element_type=f32)
        m1 = (preI[:, None, :] + preJ[None, :, :]
              + rad3 * ewr_ref[l][None] + ea3 * ewe_ref[l][None])
        m1 = silu(m1).reshape(_PP, _H)
        m2 = silu(jnp.dot(m1, eW2_ref[l], preferred_element_type=f32)
                  + eb2_ref[l])
        att = jax.nn.sigmoid(
            jnp.sum(m2 * aWr_ref[l], axis=1, keepdims=True) + ab_ref[l])
        m = m2 * att
        cm = silu(jnp.dot(m, cW1_ref[l], preferred_element_type=f32)
                  + cb1_ref[l])
        cms = jnp.tanh(jnp.sum(cm * cW2r_ref[l], axis=1, keepdims=True))
        ts3 = cms.reshape(_P, _P, 1) * inv3 * (mask_j3 * _CR)
        delta_c = jnp.concatenate(
            [jnp.sum(d3[k] * ts3, axis=1) for k in range(3)], axis=1)
        coord_c = coord_c + delta_c
        coord_r = coord_r + delta_c.T

        agg = jnp.sum((m * mask_agg).reshape(_P, _P, _H), axis=1)
        hn = silu(jnp.dot(h, nW1a_ref[l], preferred_element_type=f32)
                  + jnp.dot(agg, nW1b_ref[l], preferred_element_type=f32)
                  + nb1_ref[l])
        h = h + jnp.dot(hn, nW2_ref[l], preferred_element_type=f32) \
            + nb2_ref[l]

    vel = (coord_c - x0c)[:_N, :]
    vel = vel - jnp.sum(vel, axis=0, keepdims=True) * (1.0 / _N)
    out_ref[0] = vel


def kernel(t, x, d_base, emb_W, emb_b, edge_W1, edge_b1, edge_W2, edge_b2,
           node_W1, node_b1, node_W2, node_b2, coord_W1, coord_b1, coord_W2,
           att_W, att_b):
    B = t.shape[0]
    x3 = x.reshape(B, _N, 3)
    xt = jnp.swapaxes(x3, 1, 2)
    t3 = t.reshape(B, 1, 1)
    db3 = d_base.reshape(B, 1, 1)

    eW1a = edge_W1[:, :_H, :]
    eW1b = edge_W1[:, _H:2 * _H, :]
    ewr = edge_W1[:, 2 * _H:2 * _H + 1, :]
    ewe = edge_W1[:, 2 * _H + 1:, :]
    nW1a = node_W1[:, :_H, :]
    nW1b = node_W1[:, _H:, :]
    operands = (
        t3, db3, x3, xt, emb_W, emb_b.reshape(1, _H),
        eW1a, eW1b, ewr, ewe, edge_b1[:, None, :],
        edge_W2, edge_b2[:, None, :],
        nW1a, nW1b, node_b1[:, None, :], node_W2, node_b2[:, None, :],
        coord_W1, coord_b1[:, None, :], jnp.swapaxes(coord_W2, 1, 2),
        jnp.swapaxes(att_W, 1, 2), att_b[:, :, None],
    )

    def batched(a):
        bs = (1,) + a.shape[1:]
        return pl.BlockSpec(bs, lambda b: (b,) + (0,) * (a.ndim - 1))

    def full(a):
        return pl.BlockSpec(a.shape, lambda b: (0,) * a.ndim)

    in_specs = [batched(o) for o in operands[:4]] + \
               [full(o) for o in operands[4:]]

    out = pl.pallas_call(
        _fwd_kernel,
        grid=(B,),
        in_specs=in_specs,
        out_specs=pl.BlockSpec((1, _N, 3), lambda b: (b, 0, 0)),
        out_shape=jax.ShapeDtypeStruct((B, _N, 3), jnp.float32),
        compiler_params=pltpu.CompilerParams(
            dimension_semantics=("parallel",)),
    )(*operands)
    return out.reshape(B, _N * 3)
